# baseline (speedup 1.0000x reference)
import functools

import jax
import jax.numpy as jnp
from jax import lax
from jax.experimental import pallas as pl
from jax.experimental.pallas import tpu as pltpu
from jax.experimental.pallas import tpu_sc as plsc

F32 = jnp.float32

NC = 2
NS = 16
NW = NC * NS


def _sp(x):
    return jnp.maximum(x, 0.0) + jnp.log(1.0 + jnp.exp(-jnp.abs(x)))


def _node_emb_body(x_ref, w_ref, b_ref, o_ref, ob_ref):
    res = (
        jnp.dot(x_ref[...], w_ref[...], preferred_element_type=F32) + b_ref[...]
    )
    o_ref[...] = res
    ob_ref[...] = res.astype(jnp.bfloat16)


def _edge_mlp_body(hr_ref, hc_ref, ea_ref, we_ref, be_ref, w1a_ref, w1b_ref,
                   w1c_ref, b1_ref, w2_ref, b2_ref, o_ref):
    e = jnp.dot(ea_ref[...], we_ref[...], preferred_element_type=F32) + be_ref[...]
    z = (
        jnp.dot(hr_ref[...], w1a_ref[...], preferred_element_type=F32)
        + jnp.dot(hc_ref[...], w1b_ref[...], preferred_element_type=F32)
        + jnp.dot(e, w1c_ref[...], preferred_element_type=F32)
        + b1_ref[...]
    )
    hid = _sp(z)
    o_ref[...] = _sp(jnp.dot(hid, w2_ref[...], preferred_element_type=F32) + b2_ref[...])


def _node_update_body(h_ref, a_ref, wa_ref, wb_ref, b_ref, o_ref, ob_ref):
    aggr = a_ref[0] + a_ref[1]
    z = (
        jnp.dot(h_ref[...], wa_ref[...], preferred_element_type=F32)
        + jnp.dot(aggr, wb_ref[...], preferred_element_type=F32)
        + b_ref[...]
    )
    res = _sp(z) + h_ref[...]
    o_ref[...] = res
    ob_ref[...] = res.astype(jnp.bfloat16)


def _pool_heads_body(h_ref, bat_ref, ws1_ref, bs1_ref, ws2_ref, bs2_ref,
                     wh1_ref, bh1_ref, wh2_ref, bh2_ref, wlt_ref, blt_ref,
                     o_ref, sums, cnts, *, ngrid, nb, g):
    i = pl.program_id(0)
    bat = bat_ref[0]
    iota = lax.broadcasted_iota(jnp.int32, (g, nb), 0)
    onehot_t = jnp.where(iota == bat, 1.0, 0.0).astype(F32)
    dnum = (((1,), (0,)), ((), ()))
    s_contrib = lax.dot_general(onehot_t, h_ref[...], dnum,
                                preferred_element_type=F32)
    ones = jnp.ones((nb, o_ref.shape[1] * 2), dtype=F32)
    c_contrib = lax.dot_general(onehot_t, ones, dnum, preferred_element_type=F32)

    @pl.when(i == 0)
    def _init():
        sums[...] = s_contrib
        cnts[...] = c_contrib

    @pl.when(i > 0)
    def _acc():
        sums[...] = sums[...] + s_contrib
        cnts[...] = cnts[...] + c_contrib

    @pl.when(i == ngrid - 1)
    def _final():
        pooled = sums[...] / jnp.maximum(cnts[...], 1.0)
        s = _sp(jnp.dot(pooled, ws1_ref[...], preferred_element_type=F32) + bs1_ref[...])
        s = _sp(jnp.dot(s, ws2_ref[...], preferred_element_type=F32) + bs2_ref[...])
        for j in range(3):
            t = _sp(jnp.dot(s, wh1_ref[j], preferred_element_type=F32)
                    + bh1_ref[pl.ds(j, 1), :])
            t = _sp(jnp.dot(t, wh2_ref[j], preferred_element_type=F32)
                    + bh2_ref[pl.ds(j, 1), :])
            pred = lax.dot_general(wlt_ref[pl.ds(j, 1), :], t,
                                   (((1,), (1,)), ((), ())),
                                   preferred_element_type=F32)
            o_ref[pl.ds(j, 1), :] = pred + blt_ref[:, pl.ds(j, 1)]


def _make_gather(e, n, h, chunk, dtype=F32):
    per_w = e // NW
    iters = per_w // chunk
    assert per_w % chunk == 0
    mesh = plsc.VectorSubcoreMesh(core_axis_name="c", subcore_axis_name="s")

    @functools.partial(
        pl.kernel,
        out_type=(
            jax.ShapeDtypeStruct((e, h), dtype),
            jax.ShapeDtypeStruct((e, h), dtype),
        ),
        mesh=mesh,
        scratch_types=[
            pltpu.VMEM((2 * chunk,), jnp.int32),
            pltpu.VMEM((2 * chunk,), jnp.int32),
            pltpu.VMEM((2, 2 * chunk, h), dtype),
            pltpu.SemaphoreType.DMA,
            pltpu.SemaphoreType.DMA,
        ],
        compiler_params=pltpu.CompilerParams(use_tc_tiling_on_sc=False),
    )
    def gather_kernel(h_hbm, row_hbm, col_hbm, or_hbm, oc_hbm,
                      idx0, idx1, buf, sem0, sem1):
        cid = lax.axis_index("c")
        sid = lax.axis_index("s")
        base = (cid * NS + sid) * per_w
        sems = (sem0, sem1)
        idxs = (idx0, idx1)

        def issue(i, slot, sem):
            off = base + i * chunk
            pltpu.sync_copy(row_hbm.at[pl.ds(off, chunk)],
                            idxs[slot].at[pl.ds(0, chunk)])
            pltpu.sync_copy(col_hbm.at[pl.ds(off, chunk)],
                            idxs[slot].at[pl.ds(chunk, chunk)])
            pltpu.async_copy(h_hbm.at[idxs[slot]], buf.at[slot], sem)

        issue(0, 0, sems[0])

        def drain(i, cur):
            pltpu.make_async_copy(
                h_hbm.at[idxs[cur]], buf.at[cur], sems[cur]).wait()
            off = base + i * chunk
            pltpu.sync_copy(buf.at[cur, pl.ds(0, chunk)],
                            or_hbm.at[pl.ds(off, chunk)])
            pltpu.sync_copy(buf.at[cur, pl.ds(chunk, chunk)],
                            oc_hbm.at[pl.ds(off, chunk)])

        def pair_body(p, carry):
            for b in (0, 1):
                i = 2 * p + b
                cur, nxt = b, 1 - b

                @pl.when(i + 1 < iters)
                def _prefetch():
                    issue(i + 1, nxt, sems[nxt])

                drain(i, cur)
            return carry

        lax.fori_loop(0, iters // 2, pair_body, 0)
        if iters % 2 == 1:
            drain(iters - 1, 0)

    return gather_kernel


def _make_scatter(e, n, h, chunk):
    per_core = e // NC
    per_w = per_core // NS
    iters = per_w // chunk
    assert per_w % chunk == 0
    n_pad = ((n + 640 * NS - 1) // (640 * NS)) * (640 * NS)
    rows_per_s = n_pad // NS
    zrows = 40
    zsteps = rows_per_s // zrows
    assert zsteps * zrows == rows_per_s
    mesh = plsc.VectorSubcoreMesh(core_axis_name="c", subcore_axis_name="s")

    @functools.partial(
        pl.kernel,
        out_type=jax.ShapeDtypeStruct((NC, n_pad, h), F32),
        mesh=mesh,
        scratch_types=[
            pltpu.VMEM_SHARED((n_pad, h), F32),
            pltpu.VMEM((zrows, h), F32),
            pltpu.VMEM((chunk,), jnp.int32),
            pltpu.VMEM((chunk,), jnp.int32),
            pltpu.VMEM((2, chunk, h), F32),
            pltpu.SemaphoreType.DMA,
            pltpu.SemaphoreType.DMA,
            pltpu.SemaphoreType.DMA,
            pltpu.SemaphoreType.DMA,
        ],
    )
    def scatter_kernel(emb_hbm, col_hbm, out_hbm, acc, zbuf, idx0, idx1,
                       buf, se0, se1, si0, si1):
        cid = lax.axis_index("c")
        sid = lax.axis_index("s")
        ses = (se0, se1)
        sis = (si0, si1)
        idxs = (idx0, idx1)

        base = cid * per_core + sid * per_w

        def issue(i, slot):
            off = base + i * chunk
            pltpu.async_copy(col_hbm.at[pl.ds(off, chunk)], idxs[slot],
                             sis[slot])
            pltpu.async_copy(emb_hbm.at[pl.ds(off, chunk)], buf.at[slot],
                             ses[slot])

        def drain(i, cur):
            off = base + i * chunk
            pltpu.make_async_copy(
                col_hbm.at[pl.ds(off, chunk)], idxs[cur], sis[cur]).wait()
            pltpu.make_async_copy(
                emb_hbm.at[pl.ds(off, chunk)], buf.at[cur], ses[cur]).wait()
            pltpu.sync_copy(buf.at[cur], acc.at[idxs[cur]], add=True)

        issue(0, 0)

        zero16 = jnp.zeros((16,), dtype=F32)

        def zrow(i, carry):
            def zcol(j, c2):
                zbuf[i, pl.ds(j * 16, 16)] = zero16
                return c2
            return lax.fori_loop(0, h // 16, zcol, carry)

        lax.fori_loop(0, zrows, zrow, 0)

        def zcopy(t, carry):
            pltpu.sync_copy(zbuf, acc.at[pl.ds(sid * rows_per_s + t * zrows, zrows)])
            return carry

        lax.fori_loop(0, zsteps, zcopy, 0)
        plsc.subcore_barrier()

        def pair_body(p, carry):
            for b in (0, 1):
                i = 2 * p + b
                cur, nxt = b, 1 - b

                @pl.when(i + 1 < iters)
                def _prefetch():
                    issue(i + 1, nxt)

                drain(i, cur)
            return carry

        lax.fori_loop(0, iters // 2, pair_body, 0)
        if iters % 2 == 1:
            drain(iters - 1, 0)
        plsc.subcore_barrier()

        r0 = sid * rows_per_s
        pltpu.sync_copy(acc.at[pl.ds(r0, rows_per_s)],
                        out_hbm.at[cid, pl.ds(r0, rows_per_s)])

    return scatter_kernel


def kernel(x, edge_index, edge_attr, batch, params):
    n, d_in = x.shape
    e = edge_attr.shape[0]
    d_edge = edge_attr.shape[1]
    hdim = params["node_emb"]["W"].shape[1]
    g = 64
    nprops = len(params["heads"])

    row = edge_index[0]
    col = edge_index[1]

    nb = 2000
    ngrid_n = n // nb
    eb = 2560
    ngrid_e = e // eb

    wn = params["node_emb"]["W"]
    bn = params["node_emb"]["b"].reshape(1, hdim)
    h, hb = pl.pallas_call(
        _node_emb_body,
        grid=(ngrid_n,),
        in_specs=[
            pl.BlockSpec((nb, d_in), lambda i: (i, 0)),
            pl.BlockSpec((d_in, hdim), lambda i: (0, 0)),
            pl.BlockSpec((1, hdim), lambda i: (0, 0)),
        ],
        out_specs=[
            pl.BlockSpec((nb, hdim), lambda i: (i, 0)),
            pl.BlockSpec((nb, hdim), lambda i: (i, 0)),
        ],
        out_shape=[
            jax.ShapeDtypeStruct((n, hdim), F32),
            jax.ShapeDtypeStruct((n, hdim), jnp.bfloat16),
        ],
    )(x, wn, bn)

    we = params["edge_emb"]["W"]
    be = params["edge_emb"]["b"].reshape(1, hdim)

    hdim2 = hdim // 2
    gather_call = _make_gather(e, n, hdim2, 400, dtype=jnp.uint32)
    scatter_call = _make_scatter(e, n, hdim, 80)

    def _pack_u32(hb_arr):
        return lax.bitcast_convert_type(
            hb_arr.reshape(n, hdim2, 2), jnp.uint32)

    for cp in params["convs"]:
        w1 = cp["edge_l1"]["W"]
        w1a = w1[:hdim].astype(jnp.bfloat16)
        w1b = w1[hdim:2 * hdim].astype(jnp.bfloat16)
        w1c = w1[2 * hdim:]
        b1 = cp["edge_l1"]["b"].reshape(1, 2 * hdim)
        w2 = cp["edge_l2"]["W"]
        b2 = cp["edge_l2"]["b"].reshape(1, hdim)
        wn1 = cp["node_l1"]["W"]
        wna, wnb = wn1[:hdim], wn1[hdim:]
        bn1 = cp["node_l1"]["b"].reshape(1, hdim)

        def edge_mlp(hrow, hcol, ea):
            return pl.pallas_call(
                _edge_mlp_body,
                grid=(ngrid_e,),
                in_specs=[
                    pl.BlockSpec((eb, hdim), lambda i: (i, 0)),
                    pl.BlockSpec((eb, hdim), lambda i: (i, 0)),
                    pl.BlockSpec((eb, d_edge), lambda i: (i, 0)),
                    pl.BlockSpec((d_edge, hdim), lambda i: (0, 0)),
                    pl.BlockSpec((1, hdim), lambda i: (0, 0)),
                    pl.BlockSpec((hdim, 2 * hdim), lambda i: (0, 0)),
                    pl.BlockSpec((hdim, 2 * hdim), lambda i: (0, 0)),
                    pl.BlockSpec((hdim, 2 * hdim), lambda i: (0, 0)),
                    pl.BlockSpec((1, 2 * hdim), lambda i: (0, 0)),
                    pl.BlockSpec((2 * hdim, hdim), lambda i: (0, 0)),
                    pl.BlockSpec((1, hdim), lambda i: (0, 0)),
                ],
                out_specs=pl.BlockSpec((eb, hdim), lambda i: (i, 0)),
                out_shape=jax.ShapeDtypeStruct((e, hdim), F32),
            )(hrow, hcol, ea, we, be, w1a, w1b, w1c, b1, w2, b2)

        hrow_u, hcol_u = gather_call(_pack_u32(hb), row, col)
        hrow = lax.bitcast_convert_type(hrow_u, jnp.bfloat16).reshape(e, hdim)
        hcol = lax.bitcast_convert_type(hcol_u, jnp.bfloat16).reshape(e, hdim)
        emb = edge_mlp(hrow, hcol, edge_attr)
        aparts = scatter_call(emb, col)

        h, hb = pl.pallas_call(
            _node_update_body,
            grid=(ngrid_n,),
            in_specs=[
                pl.BlockSpec((nb, hdim), lambda i: (i, 0)),
                pl.BlockSpec((NC, nb, hdim), lambda i: (0, i, 0)),
                pl.BlockSpec((hdim, hdim), lambda i: (0, 0)),
                pl.BlockSpec((hdim, hdim), lambda i: (0, 0)),
                pl.BlockSpec((1, hdim), lambda i: (0, 0)),
            ],
            out_specs=[
                pl.BlockSpec((nb, hdim), lambda i: (i, 0)),
                pl.BlockSpec((nb, hdim), lambda i: (i, 0)),
            ],
            out_shape=[
                jax.ShapeDtypeStruct((n, hdim), F32),
                jax.ShapeDtypeStruct((n, hdim), jnp.bfloat16),
            ],
        )(h, aparts, wna, wnb, bn1)

    ws1 = params["shared"][0]["W"]
    bs1 = params["shared"][0]["b"].reshape(1, hdim)
    ws2 = params["shared"][1]["W"]
    bs2 = params["shared"][1]["b"].reshape(1, hdim)
    wh1 = jnp.stack([hd[0]["W"] for hd in params["heads"]])
    bh1 = jnp.stack([hd[0]["b"] for hd in params["heads"]])
    wh2 = jnp.stack([hd[1]["W"] for hd in params["heads"]])
    bh2 = jnp.stack([hd[1]["b"] for hd in params["heads"]])
    wlt = jnp.concatenate(
        [hd[2]["W"].reshape(1, hdim) for hd in params["heads"]]
        + [jnp.zeros((8 - nprops, hdim), F32)], axis=0)
    blt = jnp.concatenate(
        [jnp.stack([hd[2]["b"][0] for hd in params["heads"]]),
         jnp.zeros((8 - nprops,), F32)]).reshape(1, 8)

    batch3d = batch.reshape(ngrid_n, 1, nb).astype(jnp.int32)

    preds8 = pl.pallas_call(
        functools.partial(_pool_heads_body, ngrid=ngrid_n, nb=nb, g=g),
        grid=(ngrid_n,),
        in_specs=[
            pl.BlockSpec((nb, hdim), lambda i: (i, 0)),
            pl.BlockSpec((1, 1, nb), lambda i: (i, 0, 0)),
            pl.BlockSpec((hdim, hdim), lambda i: (0, 0)),
            pl.BlockSpec((1, hdim), lambda i: (0, 0)),
            pl.BlockSpec((hdim, hdim), lambda i: (0, 0)),
            pl.BlockSpec((1, hdim), lambda i: (0, 0)),
            pl.BlockSpec((3, hdim, hdim), lambda i: (0, 0, 0)),
            pl.BlockSpec((3, hdim), lambda i: (0, 0)),
            pl.BlockSpec((3, hdim, hdim), lambda i: (0, 0, 0)),
            pl.BlockSpec((3, hdim), lambda i: (0, 0)),
            pl.BlockSpec((8, hdim), lambda i: (0, 0)),
            pl.BlockSpec((1, 8), lambda i: (0, 0)),
        ],
        out_specs=pl.BlockSpec((8, g), lambda i: (0, 0)),
        out_shape=jax.ShapeDtypeStruct((8, g), F32),
        scratch_shapes=[
            pltpu.VMEM((g, hdim), F32),
            pltpu.VMEM((g, hdim), F32),
        ],
    )(h, batch3d, ws1, bs1, ws2, bs2, wh1, bh1, wh2, bh2, wlt, blt)

    return preds8[:nprops]


# speedup vs baseline: 2.7774x; 2.7774x over previous
import functools

import jax
import jax.numpy as jnp
from jax import lax
from jax.experimental import pallas as pl
from jax.experimental.pallas import tpu as pltpu
from jax.experimental.pallas import tpu_sc as plsc

F32 = jnp.float32

NC = 2
NS = 16
NW = NC * NS


def _sp(x):
    return jnp.maximum(x, 0.0) + jnp.log(1.0 + jnp.exp(-jnp.abs(x)))


def _node_emb_body(x_ref, w_ref, b_ref, o_ref):
    o_ref[...] = (
        jnp.dot(x_ref[...], w_ref[...], preferred_element_type=F32) + b_ref[...]
    )


def _edge_mlp_body(hr_ref, hc_ref, ea_ref, we_ref, be_ref, w1a_ref, w1b_ref,
                   w1c_ref, b1_ref, w2_ref, b2_ref, o_ref):
    e = jnp.dot(ea_ref[...], we_ref[...], preferred_element_type=F32) + be_ref[...]
    z = (
        jnp.dot(hr_ref[...], w1a_ref[...], preferred_element_type=F32)
        + jnp.dot(hc_ref[...], w1b_ref[...], preferred_element_type=F32)
        + jnp.dot(e, w1c_ref[...], preferred_element_type=F32)
        + b1_ref[...]
    )
    hid = _sp(z)
    o_ref[...] = _sp(jnp.dot(hid, w2_ref[...], preferred_element_type=F32) + b2_ref[...])


def _node_update_body(h_ref, a_ref, wa_ref, wb_ref, b_ref, o_ref):
    aggr = a_ref[0] + a_ref[1]
    z = (
        jnp.dot(h_ref[...], wa_ref[...], preferred_element_type=F32)
        + jnp.dot(aggr, wb_ref[...], preferred_element_type=F32)
        + b_ref[...]
    )
    o_ref[...] = _sp(z) + h_ref[...]


def _pool_heads_body(h_ref, bat_ref, ws1_ref, bs1_ref, ws2_ref, bs2_ref,
                     wh1_ref, bh1_ref, wh2_ref, bh2_ref, wlt_ref, blt_ref,
                     o_ref, sums, cnts, *, ngrid, nb, g):
    i = pl.program_id(0)
    bat = bat_ref[0]
    iota = lax.broadcasted_iota(jnp.int32, (g, nb), 0)
    onehot_t = jnp.where(iota == bat, 1.0, 0.0).astype(F32)
    dnum = (((1,), (0,)), ((), ()))
    s_contrib = lax.dot_general(onehot_t, h_ref[...], dnum,
                                preferred_element_type=F32)
    ones = jnp.ones((nb, o_ref.shape[1] * 2), dtype=F32)
    c_contrib = lax.dot_general(onehot_t, ones, dnum, preferred_element_type=F32)

    @pl.when(i == 0)
    def _init():
        sums[...] = s_contrib
        cnts[...] = c_contrib

    @pl.when(i > 0)
    def _acc():
        sums[...] = sums[...] + s_contrib
        cnts[...] = cnts[...] + c_contrib

    @pl.when(i == ngrid - 1)
    def _final():
        pooled = sums[...] / jnp.maximum(cnts[...], 1.0)
        s = _sp(jnp.dot(pooled, ws1_ref[...], preferred_element_type=F32) + bs1_ref[...])
        s = _sp(jnp.dot(s, ws2_ref[...], preferred_element_type=F32) + bs2_ref[...])
        for j in range(3):
            t = _sp(jnp.dot(s, wh1_ref[j], preferred_element_type=F32)
                    + bh1_ref[pl.ds(j, 1), :])
            t = _sp(jnp.dot(t, wh2_ref[j], preferred_element_type=F32)
                    + bh2_ref[pl.ds(j, 1), :])
            pred = lax.dot_general(wlt_ref[pl.ds(j, 1), :], t,
                                   (((1,), (1,)), ((), ())),
                                   preferred_element_type=F32)
            o_ref[pl.ds(j, 1), :] = pred + blt_ref[:, pl.ds(j, 1)]


def _make_gather(e, n, h, chunk):
    per_w = e // NW
    iters = per_w // chunk
    assert per_w % chunk == 0
    mesh = plsc.VectorSubcoreMesh(core_axis_name="c", subcore_axis_name="s")

    @functools.partial(
        pl.kernel,
        out_type=(
            jax.ShapeDtypeStruct((e, h), F32),
            jax.ShapeDtypeStruct((e, h), F32),
        ),
        mesh=mesh,
        scratch_types=[
            pltpu.VMEM((2 * chunk,), jnp.int32),
            pltpu.VMEM((2 * chunk,), jnp.int32),
            pltpu.VMEM((2, 2 * chunk, h), F32),
            pltpu.SemaphoreType.DMA,
            pltpu.SemaphoreType.DMA,
        ],
    )
    def gather_kernel(h_hbm, row_hbm, col_hbm, or_hbm, oc_hbm,
                      idx0, idx1, buf, sem0, sem1):
        cid = lax.axis_index("c")
        sid = lax.axis_index("s")
        base = (cid * NS + sid) * per_w
        sems = (sem0, sem1)
        idxs = (idx0, idx1)

        def issue(i, slot, sem):
            off = base + i * chunk
            pltpu.sync_copy(row_hbm.at[pl.ds(off, chunk)],
                            idxs[slot].at[pl.ds(0, chunk)])
            pltpu.sync_copy(col_hbm.at[pl.ds(off, chunk)],
                            idxs[slot].at[pl.ds(chunk, chunk)])
            pltpu.async_copy(h_hbm.at[idxs[slot]], buf.at[slot], sem)

        issue(0, 0, sems[0])

        def drain(i, cur):
            pltpu.make_async_copy(
                h_hbm.at[idxs[cur]], buf.at[cur], sems[cur]).wait()
            off = base + i * chunk
            pltpu.sync_copy(buf.at[cur, pl.ds(0, chunk)],
                            or_hbm.at[pl.ds(off, chunk)])
            pltpu.sync_copy(buf.at[cur, pl.ds(chunk, chunk)],
                            oc_hbm.at[pl.ds(off, chunk)])

        def pair_body(p, carry):
            for b in (0, 1):
                i = 2 * p + b
                cur, nxt = b, 1 - b

                @pl.when(i + 1 < iters)
                def _prefetch():
                    issue(i + 1, nxt, sems[nxt])

                drain(i, cur)
            return carry

        lax.fori_loop(0, iters // 2, pair_body, 0)
        if iters % 2 == 1:
            drain(iters - 1, 0)

    return gather_kernel


def _make_scatter(e, n, h, chunk):
    per_core = e // NC
    per_w = per_core // NS
    iters = per_w // chunk
    assert per_w % chunk == 0
    n_pad = ((n + 640 * NS - 1) // (640 * NS)) * (640 * NS)
    rows_per_s = n_pad // NS
    zrows = 40
    zsteps = rows_per_s // zrows
    assert zsteps * zrows == rows_per_s
    mesh = plsc.VectorSubcoreMesh(core_axis_name="c", subcore_axis_name="s")

    @functools.partial(
        pl.kernel,
        out_type=jax.ShapeDtypeStruct((NC, n_pad, h), F32),
        mesh=mesh,
        scratch_types=[
            pltpu.VMEM_SHARED((n_pad, h), F32),
            pltpu.VMEM((zrows, h), F32),
            pltpu.VMEM((chunk,), jnp.int32),
            pltpu.VMEM((chunk,), jnp.int32),
            pltpu.VMEM((2, chunk, h), F32),
            pltpu.SemaphoreType.DMA,
            pltpu.SemaphoreType.DMA,
            pltpu.SemaphoreType.DMA,
            pltpu.SemaphoreType.DMA,
        ],
    )
    def scatter_kernel(emb_hbm, col_hbm, out_hbm, acc, zbuf, idx0, idx1,
                       buf, se0, se1, si0, si1):
        cid = lax.axis_index("c")
        sid = lax.axis_index("s")
        ses = (se0, se1)
        sis = (si0, si1)
        idxs = (idx0, idx1)

        base = cid * per_core + sid * per_w

        def issue(i, slot):
            off = base + i * chunk
            pltpu.async_copy(col_hbm.at[pl.ds(off, chunk)], idxs[slot],
                             sis[slot])
            pltpu.async_copy(emb_hbm.at[pl.ds(off, chunk)], buf.at[slot],
                             ses[slot])

        def drain(i, cur):
            off = base + i * chunk
            pltpu.make_async_copy(
                col_hbm.at[pl.ds(off, chunk)], idxs[cur], sis[cur]).wait()
            pltpu.make_async_copy(
                emb_hbm.at[pl.ds(off, chunk)], buf.at[cur], ses[cur]).wait()
            pltpu.sync_copy(buf.at[cur], acc.at[idxs[cur]], add=True)

        issue(0, 0)

        zero16 = jnp.zeros((16,), dtype=F32)

        def zrow(i, carry):
            def zcol(j, c2):
                zbuf[i, pl.ds(j * 16, 16)] = zero16
                return c2
            return lax.fori_loop(0, h // 16, zcol, carry)

        lax.fori_loop(0, zrows, zrow, 0)

        def zcopy(t, carry):
            pltpu.sync_copy(zbuf, acc.at[pl.ds(sid * rows_per_s + t * zrows, zrows)])
            return carry

        lax.fori_loop(0, zsteps, zcopy, 0)
        plsc.subcore_barrier()

        def pair_body(p, carry):
            for b in (0, 1):
                i = 2 * p + b
                cur, nxt = b, 1 - b

                @pl.when(i + 1 < iters)
                def _prefetch():
                    issue(i + 1, nxt)

                drain(i, cur)
            return carry

        lax.fori_loop(0, iters // 2, pair_body, 0)
        if iters % 2 == 1:
            drain(iters - 1, 0)
        plsc.subcore_barrier()

        r0 = sid * rows_per_s
        pltpu.sync_copy(acc.at[pl.ds(r0, rows_per_s)],
                        out_hbm.at[cid, pl.ds(r0, rows_per_s)])

    return scatter_kernel


def kernel(x, edge_index, edge_attr, batch, params):
    n, d_in = x.shape
    e = edge_attr.shape[0]
    d_edge = edge_attr.shape[1]
    hdim = params["node_emb"]["W"].shape[1]
    g = 64
    nprops = len(params["heads"])

    row = edge_index[0]
    col = edge_index[1]

    nb = 2000
    ngrid_n = n // nb
    eb = 2560
    ngrid_e = e // eb

    wn = params["node_emb"]["W"]
    bn = params["node_emb"]["b"].reshape(1, hdim)
    h = pl.pallas_call(
        _node_emb_body,
        grid=(ngrid_n,),
        in_specs=[
            pl.BlockSpec((nb, d_in), lambda i: (i, 0)),
            pl.BlockSpec((d_in, hdim), lambda i: (0, 0)),
            pl.BlockSpec((1, hdim), lambda i: (0, 0)),
        ],
        out_specs=pl.BlockSpec((nb, hdim), lambda i: (i, 0)),
        out_shape=jax.ShapeDtypeStruct((n, hdim), F32),
    )(x, wn, bn)

    we = params["edge_emb"]["W"]
    be = params["edge_emb"]["b"].reshape(1, hdim)

    gather_call = _make_gather(e, n, hdim, 200)
    scatter_call = _make_scatter(e, n, hdim, 80)

    for cp in params["convs"]:
        w1 = cp["edge_l1"]["W"]
        w1a, w1b, w1c = w1[:hdim], w1[hdim:2 * hdim], w1[2 * hdim:]
        b1 = cp["edge_l1"]["b"].reshape(1, 2 * hdim)
        w2 = cp["edge_l2"]["W"]
        b2 = cp["edge_l2"]["b"].reshape(1, hdim)
        wn1 = cp["node_l1"]["W"]
        wna, wnb = wn1[:hdim], wn1[hdim:]
        bn1 = cp["node_l1"]["b"].reshape(1, hdim)

        def edge_mlp(hrow, hcol, ea):
            return pl.pallas_call(
                _edge_mlp_body,
                grid=(ngrid_e,),
                in_specs=[
                    pl.BlockSpec((eb, hdim), lambda i: (i, 0)),
                    pl.BlockSpec((eb, hdim), lambda i: (i, 0)),
                    pl.BlockSpec((eb, d_edge), lambda i: (i, 0)),
                    pl.BlockSpec((d_edge, hdim), lambda i: (0, 0)),
                    pl.BlockSpec((1, hdim), lambda i: (0, 0)),
                    pl.BlockSpec((hdim, 2 * hdim), lambda i: (0, 0)),
                    pl.BlockSpec((hdim, 2 * hdim), lambda i: (0, 0)),
                    pl.BlockSpec((hdim, 2 * hdim), lambda i: (0, 0)),
                    pl.BlockSpec((1, 2 * hdim), lambda i: (0, 0)),
                    pl.BlockSpec((2 * hdim, hdim), lambda i: (0, 0)),
                    pl.BlockSpec((1, hdim), lambda i: (0, 0)),
                ],
                out_specs=pl.BlockSpec((eb, hdim), lambda i: (i, 0)),
                out_shape=jax.ShapeDtypeStruct((e, hdim), F32),
            )(hrow, hcol, ea, we, be, w1a, w1b, w1c, b1, w2, b2)

        hrow, hcol = gather_call(h, row, col)
        emb = edge_mlp(hrow, hcol, edge_attr)
        aparts = scatter_call(emb, col)

        h = pl.pallas_call(
            _node_update_body,
            grid=(ngrid_n,),
            in_specs=[
                pl.BlockSpec((nb, hdim), lambda i: (i, 0)),
                pl.BlockSpec((NC, nb, hdim), lambda i: (0, i, 0)),
                pl.BlockSpec((hdim, hdim), lambda i: (0, 0)),
                pl.BlockSpec((hdim, hdim), lambda i: (0, 0)),
                pl.BlockSpec((1, hdim), lambda i: (0, 0)),
            ],
            out_specs=pl.BlockSpec((nb, hdim), lambda i: (i, 0)),
            out_shape=jax.ShapeDtypeStruct((n, hdim), F32),
        )(h, aparts, wna, wnb, bn1)

    ws1 = params["shared"][0]["W"]
    bs1 = params["shared"][0]["b"].reshape(1, hdim)
    ws2 = params["shared"][1]["W"]
    bs2 = params["shared"][1]["b"].reshape(1, hdim)
    wh1 = jnp.stack([hd[0]["W"] for hd in params["heads"]])
    bh1 = jnp.stack([hd[0]["b"] for hd in params["heads"]])
    wh2 = jnp.stack([hd[1]["W"] for hd in params["heads"]])
    bh2 = jnp.stack([hd[1]["b"] for hd in params["heads"]])
    wlt = jnp.concatenate(
        [hd[2]["W"].reshape(1, hdim) for hd in params["heads"]]
        + [jnp.zeros((8 - nprops, hdim), F32)], axis=0)
    blt = jnp.concatenate(
        [jnp.stack([hd[2]["b"][0] for hd in params["heads"]]),
         jnp.zeros((8 - nprops,), F32)]).reshape(1, 8)

    batch3d = batch.reshape(ngrid_n, 1, nb).astype(jnp.int32)

    preds8 = pl.pallas_call(
        functools.partial(_pool_heads_body, ngrid=ngrid_n, nb=nb, g=g),
        grid=(ngrid_n,),
        in_specs=[
            pl.BlockSpec((nb, hdim), lambda i: (i, 0)),
            pl.BlockSpec((1, 1, nb), lambda i: (i, 0, 0)),
            pl.BlockSpec((hdim, hdim), lambda i: (0, 0)),
            pl.BlockSpec((1, hdim), lambda i: (0, 0)),
            pl.BlockSpec((hdim, hdim), lambda i: (0, 0)),
            pl.BlockSpec((1, hdim), lambda i: (0, 0)),
            pl.BlockSpec((3, hdim, hdim), lambda i: (0, 0, 0)),
            pl.BlockSpec((3, hdim), lambda i: (0, 0)),
            pl.BlockSpec((3, hdim, hdim), lambda i: (0, 0, 0)),
            pl.BlockSpec((3, hdim), lambda i: (0, 0)),
            pl.BlockSpec((8, hdim), lambda i: (0, 0)),
            pl.BlockSpec((1, 8), lambda i: (0, 0)),
        ],
        out_specs=pl.BlockSpec((8, g), lambda i: (0, 0)),
        out_shape=jax.ShapeDtypeStruct((8, g), F32),
        scratch_shapes=[
            pltpu.VMEM((g, hdim), F32),
            pltpu.VMEM((g, hdim), F32),
        ],
    )(h, batch3d, ws1, bs1, ws2, bs2, wh1, bh1, wh2, bh2, wlt, blt)

    return preds8[:nprops]
